# baseline (speedup 1.0000x reference)
import functools

import jax
import jax.numpy as jnp
from jax import lax
from jax.experimental import pallas as pl
from jax.experimental.pallas import tpu as pltpu
from jax.experimental.pallas import tpu_sc as plsc

_N = 2048
_C = 8
_LANES = 128
_TILES = _N // _LANES
_SC_ROWS = 384
_TC_BLOCK = 128


def _tc_body(a_ref, m_ref, o_ref, t_ref):
    a = a_ref[...]
    m = m_ref[...]
    x = m * a
    p = x == 0.0
    th1 = jnp.tanh(0.5 * x)
    w = 0.5 * a
    ma = w * th1 + w
    th2 = jnp.tanh(0.5 * ma)
    v = jnp.where(p, jnp.float32(0.0), 0.03125 * th2 + jnp.float32(1.03125))
    t_ref[...] = jnp.sum(v, axis=1)
    t = t_ref[...]
    den = jnp.floor(t)
    num = (t - den) * jnp.float32(16.0)
    o_ref[...] = jnp.where(den > 0.0, num / den, jnp.float32(0.0))


def _tc_part(at, mt, rows):
    return pl.pallas_call(
        _tc_body,
        grid=(rows // _TC_BLOCK,),
        in_specs=[
            pl.BlockSpec((_TC_BLOCK, _C, _N), lambda i: (i, 0, 0)),
            pl.BlockSpec((_TC_BLOCK, _C, _N), lambda i: (i, 0, 0)),
        ],
        out_specs=pl.BlockSpec((_TC_BLOCK, _N), lambda i: (i, 0)),
        out_shape=jax.ShapeDtypeStruct((rows, _N), jnp.float32),
        scratch_shapes=[pltpu.VMEM((_TC_BLOCK, _N), jnp.float32)],
    )(at, mt)


def _sc_kernel_body(a_hbm, m_hbm, out_hbm, a_v, m_v, o_v):
    info = plsc.get_sparse_core_info()
    nc = info.num_cores
    rows_per = _SC_ROWS // (nc * info.num_subcores)
    wid = lax.axis_index("s") * nc + lax.axis_index("c")
    base = wid * rows_per

    row0 = _N - _SC_ROWS

    def row_body(i, _):
        r = base + i
        pltpu.sync_copy(a_hbm.at[row0 + r], a_v)
        pltpu.sync_copy(m_hbm.at[row0 + r], m_v)

        def tile_body(t, _):
            def group_body(g, _):
                numden = jnp.zeros((16,), jnp.float32)
                for c in range(_C):
                    a = a_v[t, c, pl.ds(g * 16, 16)]
                    m = m_v[t, c, pl.ds(g * 16, 16)]
                    x = a * m
                    p = x == 0.0
                    xc = jnp.minimum(jnp.maximum(x, jnp.float32(-0.7)),
                                     jnp.float32(2.7))
                    s1 = jnp.float32(5.0832401e-05)
                    for coef in (-6.5202994e-04, 2.5352717e-03,
                                 2.9453184e-04, -2.1244003e-02,
                                 -2.1482090e-06, 2.5006822e-01,
                                 4.9999875e-01):
                        s1 = s1 * xc + jnp.float32(coef)
                    ma = a * s1
                    v = jnp.float32(-9.8193937e-04)
                    for coef in (-2.5947613e-04, 1.5682308e-02,
                                 1.0312481e+00):
                        v = v * ma + jnp.float32(coef)
                    numden = numden + jnp.where(p, jnp.float32(0.0), v)
                den = numden.astype(jnp.int32).astype(jnp.float32)
                num = (numden - den) * jnp.float32(16.0)
                o_v[pl.ds(t * _LANES + g * 16, 16)] = jnp.where(
                    den > 0.0, num / den, jnp.float32(0.0))
                return 0

            lax.fori_loop(0, _LANES // 16, group_body, 0)
            return 0

        lax.fori_loop(0, _TILES, tile_body, 0)
        pltpu.sync_copy(o_v, out_hbm.at[r])
        return 0

    lax.fori_loop(0, rows_per, row_body, 0)


def _sc_part(a4, m4):
    mesh = plsc.VectorSubcoreMesh(core_axis_name="c", subcore_axis_name="s")
    k = functools.partial(
        pl.kernel,
        mesh=mesh,
        out_type=jax.ShapeDtypeStruct((_SC_ROWS, _N), jnp.float32),
        scratch_types=[
            pltpu.VMEM((_TILES, _C, _LANES), jnp.float32),
            pltpu.VMEM((_TILES, _C, _LANES), jnp.float32),
            pltpu.VMEM((_N,), jnp.float32),
        ],
    )(_sc_kernel_body)
    return k(a4, m4)


@jax.jit
def _run(adj, mask):
    n, _, c = adj.shape
    tc_rows = n - _SC_ROWS
    at = jnp.transpose(adj, (0, 2, 1))
    mt = jnp.transpose(mask, (0, 2, 1))
    a4 = jnp.transpose(adj.reshape(n, _TILES, _LANES, c), (0, 1, 3, 2))
    m4 = jnp.transpose(mask.reshape(n, _TILES, _LANES, c), (0, 1, 3, 2))

    parts = []
    if tc_rows:
        parts.append(_tc_part(at, mt, tc_rows))
    if _SC_ROWS:
        parts.append(_sc_part(a4, m4))
    if len(parts) == 1:
        return parts[0]
    return jnp.concatenate(parts, axis=0)


def kernel(adj, mask):
    return _run(adj, mask)


# speedup vs baseline: 1.3617x; 1.3617x over previous
import functools

import jax
import jax.numpy as jnp
from jax import lax
from jax.experimental import pallas as pl
from jax.experimental.pallas import tpu as pltpu
from jax.experimental.pallas import tpu_sc as plsc

_N = 2048
_C = 8
_LANES = 128
_TILES = _N // _LANES
_SC_ROWS = 128
_TC_BLOCK = 128


def _tc_body(a_ref, m_ref, o_ref, t_ref):
    a = a_ref[...]
    m = m_ref[...]
    x = m * a
    p = x == 0.0
    th1 = jnp.tanh(0.5 * x)
    w = 0.5 * a
    ma = w * th1 + w
    th2 = jnp.tanh(0.5 * ma)
    v = jnp.where(p, jnp.float32(0.0), 0.03125 * th2 + jnp.float32(1.03125))
    t_ref[...] = jnp.sum(v, axis=1)
    t = t_ref[...]
    den = jnp.floor(t)
    num = (t - den) * jnp.float32(16.0)
    o_ref[...] = jnp.where(den > 0.0, num / den, jnp.float32(0.0))


def _tc_part(at, mt, rows):
    return pl.pallas_call(
        _tc_body,
        grid=(rows // _TC_BLOCK,),
        in_specs=[
            pl.BlockSpec((_TC_BLOCK, _C, _N), lambda i: (i, 0, 0)),
            pl.BlockSpec((_TC_BLOCK, _C, _N), lambda i: (i, 0, 0)),
        ],
        out_specs=pl.BlockSpec((_TC_BLOCK, _N), lambda i: (i, 0)),
        out_shape=jax.ShapeDtypeStruct((rows, _N), jnp.float32),
        scratch_shapes=[pltpu.VMEM((_TC_BLOCK, _N), jnp.float32)],
    )(at, mt)


def _sc_kernel_body(a_hbm, m_hbm, out_hbm, a_v, m_v, o_v):
    info = plsc.get_sparse_core_info()
    nc = info.num_cores
    rows_per = _SC_ROWS // (nc * info.num_subcores)
    wid = lax.axis_index("s") * nc + lax.axis_index("c")
    base = wid * rows_per

    row0 = _N - _SC_ROWS

    def row_body(i, _):
        r = base + i
        pltpu.sync_copy(a_hbm.at[row0 + r], a_v)
        pltpu.sync_copy(m_hbm.at[row0 + r], m_v)

        def tile_body(t, _):
            def group_body(g, _):
                numden = jnp.zeros((16,), jnp.float32)
                for c in range(_C):
                    a = a_v[t, c, pl.ds(g * 16, 16)]
                    m = m_v[t, c, pl.ds(g * 16, 16)]
                    x = a * m
                    p = x == 0.0
                    s1 = 1.0 / (1.0 + jnp.exp(-x))
                    ma = a * s1
                    s2 = 1.0 / (1.0 + jnp.exp(-ma))
                    v = 0.0625 * s2 + jnp.float32(1.0)
                    numden = numden + jnp.where(p, jnp.float32(0.0), v)
                den = numden.astype(jnp.int32).astype(jnp.float32)
                num = (numden - den) * jnp.float32(16.0)
                o_v[pl.ds(t * _LANES + g * 16, 16)] = jnp.where(
                    den > 0.0, num / den, jnp.float32(0.0))
                return 0

            lax.fori_loop(0, _LANES // 16, group_body, 0)
            return 0

        lax.fori_loop(0, _TILES, tile_body, 0)
        pltpu.sync_copy(o_v, out_hbm.at[r])
        return 0

    lax.fori_loop(0, rows_per, row_body, 0)


def _sc_part(a4, m4):
    mesh = plsc.VectorSubcoreMesh(core_axis_name="c", subcore_axis_name="s")
    k = functools.partial(
        pl.kernel,
        mesh=mesh,
        out_type=jax.ShapeDtypeStruct((_SC_ROWS, _N), jnp.float32),
        scratch_types=[
            pltpu.VMEM((_TILES, _C, _LANES), jnp.float32),
            pltpu.VMEM((_TILES, _C, _LANES), jnp.float32),
            pltpu.VMEM((_N,), jnp.float32),
        ],
    )(_sc_kernel_body)
    return k(a4, m4)


@jax.jit
def _run(adj, mask):
    n, _, c = adj.shape
    tc_rows = n - _SC_ROWS
    at = jnp.transpose(adj, (0, 2, 1))
    mt = jnp.transpose(mask, (0, 2, 1))
    a4 = jnp.transpose(adj.reshape(n, _TILES, _LANES, c), (0, 1, 3, 2))
    m4 = jnp.transpose(mask.reshape(n, _TILES, _LANES, c), (0, 1, 3, 2))

    parts = []
    if tc_rows:
        parts.append(_tc_part(at, mt, tc_rows))
    if _SC_ROWS:
        parts.append(_sc_part(a4, m4))
    if len(parts) == 1:
        return parts[0]
    return jnp.concatenate(parts, axis=0)


def kernel(adj, mask):
    return _run(adj, mask)
